# Initial kernel scaffold; baseline (speedup 1.0000x reference)
import jax
import jax.numpy as jnp
from jax.experimental import pallas as pl


def kernel(z, pos, batch, ptr, edge_index, emb_table, W_init, b_init, fc0_W, fc0_b, W_msg, W_rbf, W_agg, fc_W, fc_b, W_final, b_final):
    raise NotImplementedError("write your pallas kernel here")


# speedup vs baseline: 2.8138x; 2.8138x over previous
import functools

import jax
import jax.numpy as jnp
from jax import lax
from jax.experimental import pallas as pl
from jax.experimental.pallas import tpu as pltpu
from jax.experimental.pallas import tpu_sc as plsc

CUTOFF = 5.0
ENV_EXP = 5


def _gelu(x):
    return jax.nn.gelu(x)


def _init_body(z_ref, emb_ref, wi_ref, bi_ref, f0w_ref, f0b_ref, wm_ref,
               x_out, xm_out, *, ncls):
    z = z_ref[...]
    ai = z[:, 0:1].astype(jnp.int32)
    oh = (ai == lax.broadcasted_iota(jnp.int32, (1, ncls), 1)).astype(jnp.float32)
    atd = emb_ref.shape[1]
    ew = emb_ref[...] @ wi_ref[0:atd, :]
    x = oh @ ew + z[:, 1:] @ wi_ref[atd:, :] + bi_ref[...]
    for l in range(f0w_ref.shape[0]):
        x = _gelu(x @ f0w_ref[l] + f0b_ref[l])
    x_out[...] = x
    xm_out[...] = x @ wm_ref[...]


def _tc_init(z, emb_table, W_init, b_init, fc0_W, fc0_b, Wm0):
    n, inw = z.shape
    d = W_init.shape[1]
    m = Wm0.shape[1]
    ncls = emb_table.shape[0]
    bn = 1000
    grid = n // bn
    return pl.pallas_call(
        functools.partial(_init_body, ncls=ncls),
        grid=(grid,),
        in_specs=[
            pl.BlockSpec((bn, inw), lambda i: (i, 0)),
            pl.BlockSpec(emb_table.shape, lambda i: (0, 0)),
            pl.BlockSpec(W_init.shape, lambda i: (0, 0)),
            pl.BlockSpec(b_init.shape, lambda i: (0,)),
            pl.BlockSpec(fc0_W.shape, lambda i: (0, 0, 0)),
            pl.BlockSpec(fc0_b.shape, lambda i: (0, 0)),
            pl.BlockSpec(Wm0.shape, lambda i: (0, 0)),
        ],
        out_specs=[
            pl.BlockSpec((bn, d), lambda i: (i, 0)),
            pl.BlockSpec((bn, m), lambda i: (i, 0)),
        ],
        out_shape=[
            jax.ShapeDtypeStruct((n, d), jnp.float32),
            jax.ShapeDtypeStruct((n, m), jnp.float32),
        ],
    )(z, emb_table, W_init, b_init, fc0_W, fc0_b, Wm0)


def _rbf_body(d2_ref, wr_ref, *outs, e_valid, be, r):
    dist = jnp.sqrt(d2_ref[...])
    x = dist / CUTOFF
    xs = jnp.maximum(x, 1e-8)
    p = ENV_EXP + 1
    a = -(p + 1) * (p + 2) / 2.0
    b = p * (p + 2)
    c = -p * (p + 1) / 2.0
    xs2 = xs * xs
    xs4 = xs2 * xs2
    xs5 = xs4 * xs
    env = (1.0 / xs + a * xs5 + b * xs5 * xs + c * xs5 * xs2) * (x < 1.0)
    freq = (lax.broadcasted_iota(jnp.int32, (1, r), 1).astype(jnp.float32)
            + 1.0) * jnp.pi
    rbf = env * jnp.sin(freq * xs)
    rid = pl.program_id(0) * be + lax.broadcasted_iota(jnp.int32, (be, 1), 0)
    rbf = jnp.where(rid < e_valid, rbf, 0.0)
    for n, o in enumerate(outs):
        o[...] = rbf @ wr_ref[n]


def _tc_rbf_rw(d2, W_rbf, e_valid):
    epad = d2.shape[0]
    nl, r, m = W_rbf.shape
    be = 2048
    grid = epad // be
    outs = pl.pallas_call(
        functools.partial(_rbf_body, e_valid=e_valid, be=be, r=r),
        grid=(grid,),
        in_specs=[
            pl.BlockSpec((be, 1), lambda i: (i, 0)),
            pl.BlockSpec(W_rbf.shape, lambda i: (0, 0, 0)),
        ],
        out_specs=[pl.BlockSpec((be, m), lambda i: (i, 0))] * nl,
        out_shape=[jax.ShapeDtypeStruct((epad, m), jnp.float32)] * nl,
    )(d2, W_rbf)
    return outs


def _update_body(x_ref, parts_ref, wa_ref, fw_ref, fb_ref, wn_ref,
                 x_out, xm_out):
    agg = parts_ref[0] + parts_ref[1]
    x1 = x_ref[...] + agg @ wa_ref[...]
    h = x1
    for l in range(fw_ref.shape[0]):
        h = _gelu(h @ fw_ref[l] + fb_ref[l])
    x2 = x1 + h
    x_out[...] = x2
    xm_out[...] = x2 @ wn_ref[...]


def _tc_update(x, parts, Wa, fW, fb, Wnext):
    n, d = x.shape
    m = parts.shape[2]
    mn = Wnext.shape[1]
    bn = 1000
    grid = n // bn
    return pl.pallas_call(
        _update_body,
        grid=(grid,),
        in_specs=[
            pl.BlockSpec((bn, d), lambda i: (i, 0)),
            pl.BlockSpec((2, bn, m), lambda i: (0, i, 0)),
            pl.BlockSpec(Wa.shape, lambda i: (0, 0)),
            pl.BlockSpec(fW.shape, lambda i: (0, 0, 0)),
            pl.BlockSpec(fb.shape, lambda i: (0, 0)),
            pl.BlockSpec(Wnext.shape, lambda i: (0, 0)),
        ],
        out_specs=[
            pl.BlockSpec((bn, d), lambda i: (i, 0)),
            pl.BlockSpec((bn, mn), lambda i: (i, 0)),
        ],
        out_shape=[
            jax.ShapeDtypeStruct((n, d), jnp.float32),
            jax.ShapeDtypeStruct((n, mn), jnp.float32),
        ],
    )(x, parts, Wa, fW, fb, Wnext)


def _final_body(x_ref, parts_ref, wa_ref, fw_ref, fb_ref, wf_ref, bf_ref,
                out_ref):
    agg = parts_ref[0] + parts_ref[1]
    x1 = x_ref[...] + agg @ wa_ref[...]
    h = x1
    for l in range(fw_ref.shape[0]):
        h = _gelu(h @ fw_ref[l] + fb_ref[l])
    x2 = x1 + h
    out_ref[...] = x2 @ wf_ref[...] + bf_ref[...]


def _tc_final(x, parts, Wa, fW, fb, Wf, bf):
    n, d = x.shape
    m = parts.shape[2]
    emb = Wf.shape[1]
    bn = 1000
    grid = n // bn
    return pl.pallas_call(
        _final_body,
        grid=(grid,),
        in_specs=[
            pl.BlockSpec((bn, d), lambda i: (i, 0)),
            pl.BlockSpec((2, bn, m), lambda i: (0, i, 0)),
            pl.BlockSpec(Wa.shape, lambda i: (0, 0)),
            pl.BlockSpec(fW.shape, lambda i: (0, 0, 0)),
            pl.BlockSpec(fb.shape, lambda i: (0, 0)),
            pl.BlockSpec(Wf.shape, lambda i: (0, 0)),
            pl.BlockSpec(bf.shape, lambda i: (0,)),
        ],
        out_specs=pl.BlockSpec((bn, emb), lambda i: (i, 0)),
        out_shape=jax.ShapeDtypeStruct((n, emb), jnp.float32),
    )(x, parts, Wa, fW, fb, Wf, bf)


_CH = 128


def _sc_edge_d2(pos_flat, ei):
    nf = pos_flat.shape[0]
    epad = ei.shape[1]
    per_w = epad // _CH // 32
    mesh = plsc.VectorSubcoreMesh(core_axis_name="c", subcore_axis_name="s")

    @functools.partial(
        pl.kernel,
        out_type=jax.ShapeDtypeStruct((epad,), jnp.float32),
        mesh=mesh,
        compiler_params=pltpu.CompilerParams(needs_layout_passes=False),
        scratch_types=[
            pltpu.VMEM((nf,), jnp.float32),
            pltpu.VMEM((_CH,), jnp.int32),
            pltpu.VMEM((_CH,), jnp.int32),
            pltpu.VMEM((_CH,), jnp.float32),
        ],
    )
    def k(pos_hbm, ei_hbm, d2_hbm, pos_v, idx_i, idx_j, d2v):
        c = lax.axis_index("c")
        s = lax.axis_index("s")
        wid = s * 2 + c
        base0 = wid * per_w * _CH
        pltpu.sync_copy(pos_hbm, pos_v)

        def body(t, _):
            base = base0 + t * _CH
            pltpu.sync_copy(ei_hbm.at[0, pl.ds(base, _CH)], idx_i)
            pltpu.sync_copy(ei_hbm.at[1, pl.ds(base, _CH)], idx_j)
            for g in range(_CH // 16):
                i16 = idx_i[pl.ds(g * 16, 16)] * 3
                j16 = idx_j[pl.ds(g * 16, 16)] * 3
                acc = jnp.zeros((16,), jnp.float32)
                for kk in range(3):
                    a = plsc.load_gather(pos_v, [i16 + kk])
                    b = plsc.load_gather(pos_v, [j16 + kk])
                    dd = a - b
                    acc = acc + dd * dd
                d2v[pl.ds(g * 16, 16)] = acc
            pltpu.sync_copy(d2v, d2_hbm.at[pl.ds(base, _CH)])
            return 0

        lax.fori_loop(0, per_w, body, 0)

    return k(pos_flat, ei)


def _sc_gather_mul_scatter(xm, rw, eic, ec):
    n, m = xm.shape
    nchunks = eic.shape[0] // (2 * ec)
    cps = nchunks // 2 // 16
    zrows = 40
    nrc = n // zrows
    nrc_full = nrc // 16
    nrc_rem = nrc - nrc_full * 16
    mesh = plsc.VectorSubcoreMesh(core_axis_name="c", subcore_axis_name="s")

    @functools.partial(
        pl.kernel,
        out_type=jax.ShapeDtypeStruct((2, n, m), jnp.float32),
        mesh=mesh,
        compiler_params=pltpu.CompilerParams(needs_layout_passes=False),
        scratch_types=[
            pltpu.VMEM((2 * ec,), jnp.int32),
            pltpu.VMEM((2 * ec,), jnp.int32),
            pltpu.VMEM((ec, m), jnp.float32),
            pltpu.VMEM((ec, m), jnp.float32),
            pltpu.VMEM((ec, m), jnp.float32),
            pltpu.VMEM((ec, m), jnp.float32),
            pltpu.VMEM((zrows, m), jnp.float32),
            pltpu.VMEM_SHARED((n, m), jnp.float32),
            pltpu.SemaphoreType.DMA,
            pltpu.SemaphoreType.DMA,
            pltpu.SemaphoreType.DMA,
            pltpu.SemaphoreType.DMA,
            pltpu.SemaphoreType.DMA,
            pltpu.SemaphoreType.DMA,
            pltpu.SemaphoreType.DMA,
            pltpu.SemaphoreType.DMA,
        ],
    )
    def k(xm_hbm, rw_hbm, ei_hbm, out_hbm, ix0, ix1, gx0, gx1,
          rw0, rw1, zb, agg, si0, si1, sg0, sg1, sr0, sr1, ss0, ss1):
        c = lax.axis_index("c")
        s = lax.axis_index("s")
        ixb = (ix0, ix1)
        gxb, rwb = (gx0, gx1), (rw0, rw1)
        si, sg, sr, ss = (si0, si1), (sg0, sg1), (sr0, sr1), (ss0, ss1)
        zv = jnp.zeros((16,), jnp.float32)
        for rr in range(zrows):
            for cc in range(m // 16):
                zb[rr, pl.ds(cc * 16, 16)] = zv

        def zbody(t, _):
            pltpu.sync_copy(zb, agg.at[pl.ds((t * 16 + s) * zrows, zrows)])
            return 0

        lax.fori_loop(0, nrc_full, zbody, 0)

        @pl.when(s < nrc_rem)
        def _():
            pltpu.sync_copy(zb, agg.at[pl.ds((nrc_full * 16 + s) * zrows, zrows)])

        plsc.subcore_barrier()

        cid0 = (c * 16 + s) * cps

        def idx_issue(t, b):
            pltpu.async_copy(ei_hbm.at[pl.ds((cid0 + t) * 2 * ec, 2 * ec)],
                             ixb[b], si[b])

        def gather_issue(t, b):
            for g in range(ec // 16):
                j16 = ixb[b][pl.ds(ec + g * 16, 16)]
                pltpu.async_copy(xm_hbm.at[j16],
                                 gxb[b].at[pl.ds(g * 16, 16)], sg[b])
            pltpu.async_copy(rw_hbm.at[pl.ds((cid0 + t) * ec, ec)],
                             rwb[b], sr[b])

        def mul(b):
            def mrow(r, _):
                for cc in range(m // 16):
                    sl = pl.ds(cc * 16, 16)
                    gxb[b][r, sl] = gxb[b][r, sl] * rwb[b][r, sl]
                return 0

            lax.fori_loop(0, ec, mrow, 0)

        def scatter_issue(b):
            for g in range(ec // 16):
                i16 = ixb[b][pl.ds(g * 16, 16)]
                pltpu.async_copy(gxb[b].at[pl.ds(g * 16, 16)],
                                 agg.at[i16], ss[b], add=True)

        def scatter_wait(b):
            for g in range(ec // 16):
                i16 = ixb[b][pl.ds(g * 16, 16)]
                pltpu.make_async_copy(gxb[b].at[pl.ds(g * 16, 16)],
                                      agg.at[i16], ss[b]).wait()

        idx_issue(0, 0)
        idx_issue(1, 1)
        pltpu.make_async_copy(ei_hbm.at[pl.ds(0, 2 * ec)], ixb[0],
                              si[0]).wait()
        gather_issue(0, 0)

        def body(t2, _):
            for b in range(2):
                t = t2 * 2 + b
                nb = 1 - b

                @pl.when(t + 1 < cps)
                def _():
                    pltpu.make_async_copy(
                        ei_hbm.at[pl.ds(0, 2 * ec)], ixb[nb],
                        si[nb]).wait()

                    @pl.when(t >= 1)
                    def _():
                        scatter_wait(nb)

                    gather_issue(t + 1, nb)

                for g in range(ec // 16):
                    j16 = ixb[b][pl.ds(ec + g * 16, 16)]
                    pltpu.make_async_copy(
                        xm_hbm.at[j16], gxb[b].at[pl.ds(g * 16, 16)],
                        sg[b]).wait()
                pltpu.make_async_copy(rw_hbm.at[pl.ds(0, ec)], rwb[b],
                                      sr[b]).wait()
                mul(b)
                scatter_issue(b)

                @pl.when(t + 2 < cps)
                def _():
                    idx_issue(t + 2, b)

            return 0

        lax.fori_loop(0, cps // 2, body, 0)
        for b in range(2):
            scatter_wait(b)
        plsc.subcore_barrier()

        def obody(t, _):
            rb = (t * 16 + s) * zrows
            pltpu.sync_copy(agg.at[pl.ds(rb, zrows)],
                            out_hbm.at[c, pl.ds(rb, zrows)])
            return 0

        lax.fori_loop(0, nrc_full, obody, 0)

        @pl.when(s < nrc_rem)
        def _():
            rb = (nrc_full * 16 + s) * zrows
            pltpu.sync_copy(agg.at[pl.ds(rb, zrows)],
                            out_hbm.at[c, pl.ds(rb, zrows)])

    return k(xm, rw, eic)


def kernel(z, pos, batch, ptr, edge_index, emb_table, W_init, b_init,
           fc0_W, fc0_b, W_msg, W_rbf, W_agg, fc_W, fc_b, W_final, b_final):
    e = edge_index.shape[1]
    nl = W_msg.shape[0]
    epad = -(-e // (32 * _CH)) * (32 * _CH)
    ei = jnp.pad(edge_index, ((0, 0), (0, epad - e)))

    pos_flat = pos.reshape(-1)
    nf = pos_flat.shape[0]
    nfp = -(-nf // 128) * 128
    pos_flat = jnp.pad(pos_flat, (0, nfp - nf))
    d2 = _sc_edge_d2(pos_flat, ei)
    x, xm = _tc_init(z, emb_table, W_init, b_init, fc0_W, fc0_b, W_msg[0])
    rws = _tc_rbf_rw(d2.reshape(-1, 1), W_rbf, e)

    ec = 64
    eic = ei.reshape(2, epad // ec, ec).transpose(1, 0, 2).reshape(-1)
    for layer in range(nl):
        parts = _sc_gather_mul_scatter(xm, rws[layer], eic, ec)
        if layer < nl - 1:
            x, xm = _tc_update(x, parts, W_agg[layer], fc_W[layer],
                               fc_b[layer], W_msg[layer + 1])
        else:
            out = _tc_final(x, parts, W_agg[layer], fc_W[layer], fc_b[layer],
                            W_final, b_final)
    return out
